# baseline (speedup 1.0000x reference)
import functools

import jax
import jax.numpy as jnp
from jax import lax
from jax.experimental import pallas as pl
from jax.experimental.pallas import tpu as pltpu
from jax.experimental.pallas import tpu_sc as plsc

B, T, D = 4096, 50, 128
NC, NS = 2, 16
NW = NC * NS
BPW = B // NW

BLK = 128
GRID = B // BLK


def _sc_gather_body(idx_hbm, table_hbm, emb_hbm, idx_v, rows_v, sem):
    wid = lax.axis_index("s") * NC + lax.axis_index("c")
    base = wid * BPW
    pltpu.sync_copy(idx_hbm.at[pl.ds(base, BPW)], idx_v)
    pltpu.async_copy(table_hbm.at[idx_v], rows_v, sem).wait()
    pltpu.sync_copy(rows_v, emb_hbm.at[pl.ds(base, BPW)])


def _tc_concat_body(emb_ref, feat_ref, out_ref):
    out_ref[:, 0, :] = emb_ref[...]
    out_ref[:, 1:, :] = feat_ref[...]


@jax.jit
def _run(feature, idx, table):
    mesh = plsc.VectorSubcoreMesh(core_axis_name="c", subcore_axis_name="s")
    emb = functools.partial(
        pl.kernel,
        out_type=jax.ShapeDtypeStruct((B, D), jnp.float32),
        mesh=mesh,
        scratch_types=[
            pltpu.VMEM((BPW,), jnp.int32),
            pltpu.VMEM((BPW, D), jnp.float32),
            pltpu.SemaphoreType.DMA,
        ],
    )(_sc_gather_body)(idx, table)

    return pl.pallas_call(
        _tc_concat_body,
        grid=(GRID,),
        in_specs=[
            pl.BlockSpec((BLK, D), lambda i: (i, 0)),
            pl.BlockSpec((BLK, T, D), lambda i: (i, 0, 0)),
        ],
        out_specs=pl.BlockSpec((BLK, T + 1, D), lambda i: (i, 0, 0)),
        out_shape=jax.ShapeDtypeStruct((B, T + 1, D), jnp.float32),
    )(emb, feature)


def kernel(feature, index_value, embedding_table):
    idx = index_value.astype(jnp.int32)
    return _run(feature, idx, embedding_table)


# speedup vs baseline: 1.0381x; 1.0381x over previous
import functools

import jax
import jax.numpy as jnp
from jax import lax
from jax.experimental import pallas as pl
from jax.experimental.pallas import tpu as pltpu
from jax.experimental.pallas import tpu_sc as plsc

B, T, D = 4096, 50, 128
NC, NS = 2, 16
NW = NC * NS
BPW = B // NW

C = 128
NCH = B // C
K = 4


def _sc_gather_body(idx_hbm, table_hbm, emb_hbm, idx_v, rows_v, sem):
    wid = lax.axis_index("s") * NC + lax.axis_index("c")
    base = wid * BPW
    pltpu.sync_copy(idx_hbm.at[pl.ds(base, BPW)], idx_v)
    pltpu.async_copy(table_hbm.at[idx_v], rows_v, sem).wait()
    pltpu.sync_copy(rows_v, emb_hbm.at[pl.ds(base, BPW)])


def _tc_concat_body(emb_hbm, feat_hbm, out_hbm,
                    feat_buf, emb_buf, out_buf,
                    in_sems, emb_sems, out_sems):
    def in_copies(g, slot):
        return (
            pltpu.make_async_copy(
                feat_hbm.at[pl.ds(g * C, C)], feat_buf.at[slot],
                in_sems.at[slot]),
            pltpu.make_async_copy(
                emb_hbm.at[pl.ds(g * C, C)], emb_buf.at[slot],
                emb_sems.at[slot]),
        )

    def out_copy(g, slot):
        return pltpu.make_async_copy(
            out_buf.at[slot], out_hbm.at[pl.ds(g * C, C)],
            out_sems.at[slot])

    for g in range(K):
        for c in in_copies(g, g):
            c.start()

    def step(g, _):
        slot = lax.rem(g, K)
        for c in in_copies(g, slot):
            c.wait()

        @pl.when(g >= K)
        def _():
            out_copy(g - K, slot).wait()

        out_buf[slot, :, 0, :] = emb_buf[slot]
        out_buf[slot, :, 1:, :] = feat_buf[slot]
        out_copy(g, slot).start()

        @pl.when(g + K < NCH)
        def _():
            for c in in_copies(g + K, slot):
                c.start()
        return 0

    lax.fori_loop(0, NCH, step, 0)
    for t in range(NCH - K, NCH):
        out_copy(t, t % K).wait()


@jax.jit
def _run(feature, idx, table):
    mesh = plsc.VectorSubcoreMesh(core_axis_name="c", subcore_axis_name="s")
    emb = functools.partial(
        pl.kernel,
        out_type=jax.ShapeDtypeStruct((B, D), jnp.float32),
        mesh=mesh,
        scratch_types=[
            pltpu.VMEM((BPW,), jnp.int32),
            pltpu.VMEM((BPW, D), jnp.float32),
            pltpu.SemaphoreType.DMA,
        ],
    )(_sc_gather_body)(idx, table)

    return pl.pallas_call(
        _tc_concat_body,
        in_specs=[
            pl.BlockSpec(memory_space=pl.ANY),
            pl.BlockSpec(memory_space=pl.ANY),
        ],
        out_specs=pl.BlockSpec(memory_space=pl.ANY),
        out_shape=jax.ShapeDtypeStruct((B, T + 1, D), jnp.float32),
        scratch_shapes=[
            pltpu.VMEM((K, C, T, D), jnp.float32),
            pltpu.VMEM((K, C, D), jnp.float32),
            pltpu.VMEM((K, C, T + 1, D), jnp.float32),
            pltpu.SemaphoreType.DMA((K,)),
            pltpu.SemaphoreType.DMA((K,)),
            pltpu.SemaphoreType.DMA((K,)),
        ],
    )(emb, feature)


def kernel(feature, index_value, embedding_table):
    idx = index_value.astype(jnp.int32)
    return _run(feature, idx, embedding_table)


# speedup vs baseline: 1.0387x; 1.0005x over previous
import functools

import jax
import jax.numpy as jnp
from jax import lax
from jax.experimental import pallas as pl
from jax.experimental.pallas import tpu as pltpu
from jax.experimental.pallas import tpu_sc as plsc

B, T, D = 4096, 50, 128
NC, NS = 2, 16
NW = NC * NS
BPW = B // NW

C = 128
NCH = B // C
K = 4


def _sc_gather_body(idx_hbm, table_hbm, emb_hbm, idx_v, rows_v, sem):
    wid = lax.axis_index("s") * NC + lax.axis_index("c")
    base = wid * BPW
    pltpu.sync_copy(idx_hbm.at[pl.ds(base, BPW)], idx_v)
    pltpu.async_copy(table_hbm.at[idx_v], rows_v, sem).wait()
    pltpu.sync_copy(rows_v, emb_hbm.at[pl.ds(base, BPW)])


def _tc_concat_body(emb_hbm, feat_hbm, out_hbm,
                    feat_buf, emb_buf, out_buf,
                    in_sems, emb_sems, out_sems):
    def in_copies(g, slot):
        return (
            pltpu.make_async_copy(
                feat_hbm.at[pl.ds(g * C, C)], feat_buf.at[slot],
                in_sems.at[slot]),
            pltpu.make_async_copy(
                emb_hbm.at[pl.ds(g * C, C)], emb_buf.at[slot],
                emb_sems.at[slot]),
        )

    def out_copy(g, slot):
        return pltpu.make_async_copy(
            out_buf.at[slot], out_hbm.at[pl.ds(g * C, C)],
            out_sems.at[slot])

    for g in range(K):
        for c in in_copies(g, g):
            c.start()

    for g in range(NCH):
        slot = g % K
        for c in in_copies(g, slot):
            c.wait()
        if g >= K:
            out_copy(g - K, slot).wait()
        out_buf[slot, :, 0, :] = emb_buf[slot]
        out_buf[slot, :, 1:, :] = feat_buf[slot]
        out_copy(g, slot).start()
        if g + K < NCH:
            for c in in_copies(g + K, slot):
                c.start()

    for t in range(NCH - K, NCH):
        out_copy(t, t % K).wait()


@jax.jit
def _run(feature, idx, table):
    mesh = plsc.VectorSubcoreMesh(core_axis_name="c", subcore_axis_name="s")
    emb = functools.partial(
        pl.kernel,
        out_type=jax.ShapeDtypeStruct((B, D), jnp.float32),
        mesh=mesh,
        scratch_types=[
            pltpu.VMEM((BPW,), jnp.int32),
            pltpu.VMEM((BPW, D), jnp.float32),
            pltpu.SemaphoreType.DMA,
        ],
    )(_sc_gather_body)(idx, table)

    return pl.pallas_call(
        _tc_concat_body,
        in_specs=[
            pl.BlockSpec(memory_space=pl.ANY),
            pl.BlockSpec(memory_space=pl.ANY),
        ],
        out_specs=pl.BlockSpec(memory_space=pl.ANY),
        out_shape=jax.ShapeDtypeStruct((B, T + 1, D), jnp.float32),
        scratch_shapes=[
            pltpu.VMEM((K, C, T, D), jnp.float32),
            pltpu.VMEM((K, C, D), jnp.float32),
            pltpu.VMEM((K, C, T + 1, D), jnp.float32),
            pltpu.SemaphoreType.DMA((K,)),
            pltpu.SemaphoreType.DMA((K,)),
            pltpu.SemaphoreType.DMA((K,)),
        ],
    )(emb, feature)


def kernel(feature, index_value, embedding_table):
    idx = index_value.astype(jnp.int32)
    return _run(feature, idx, embedding_table)
